# baseline (speedup 1.0000x reference)
import functools

import jax
import jax.numpy as jnp
from jax import lax
from jax.experimental import pallas as pl
from jax.experimental.pallas import tpu as pltpu
from jax.experimental.pallas import tpu_sc as plsc

_R = 25600
_CHUNK = 128
_C = _R // _CHUNK


def _prefix_body(nb, xt_ref, et_ref, w1t_ref, b1_ref, w2t_ref, b2_ref,
                 w3t_ref, b3_ref, out_ref, carry_ref, cs_ref):
    b = pl.program_id(0)

    @pl.when(b == 0)
    def _():
        carry_ref[...] = jnp.zeros_like(carry_ref)

    xt = xt_ref[...]
    et = et_ref[...]
    nf = xt.shape[0]
    w1at = w1t_ref[:, 0:nf]
    w1bt = w1t_ref[:, nf:]
    h1 = jnp.maximum(
        jnp.dot(w1at, xt, preferred_element_type=jnp.float32)
        + jnp.dot(w1bt, et, preferred_element_type=jnp.float32)
        + b1_ref[...], 0.0)
    h2 = jnp.maximum(
        jnp.dot(w2t_ref[...], h1, preferred_element_type=jnp.float32)
        + b2_ref[...], 0.0)
    h3 = (jnp.dot(w3t_ref[...], h2, preferred_element_type=jnp.float32)
          + b3_ref[...])

    ri = lax.broadcasted_iota(jnp.int32, (_CHUNK, _CHUNK), 0)
    ci = lax.broadcasted_iota(jnp.int32, (_CHUNK, _CHUNK), 1)
    upper = (ri < ci).astype(jnp.float32)

    for c in range(_C):
        chunk = h3[:, c * _CHUNK:(c + 1) * _CHUNK]
        exc = jnp.dot(chunk, upper, preferred_element_type=jnp.float32)
        out_ref[:, c, :] = exc
        cs_ref[:, c:c + 1] = (exc[:, _CHUNK - 1:_CHUNK]
                              + chunk[:, _CHUNK - 1:_CHUNK])
    cs = cs_ref[...]

    rc = lax.broadcasted_iota(jnp.int32, (_C, _C), 0)
    cc = lax.broadcasted_iota(jnp.int32, (_C, _C), 1)
    upc = (rc < cc).astype(jnp.float32)
    coff = (jnp.dot(cs, upc, preferred_element_type=jnp.float32)
            + carry_ref[:, 0:1])

    for c in range(_C):
        out_ref[:, c, :] += coff[:, c:c + 1]
    carry_ref[:, 0:1] = coff[:, _C - 1:_C] + cs[:, _C - 1:_C]


def _prefix_call(xt, et, w1t, b1c, w2t, b2c, w3t, b3c, interpret=False):
    nf, n = xt.shape
    de = et.shape[0]
    d1 = w1t.shape[0]
    d2 = w2t.shape[0]
    do = w3t.shape[0]
    nb = n // _R
    assert n % _R == 0
    body = functools.partial(_prefix_body, nb)

    def in_idx(b):
        return (0, jnp.minimum(b, nb - 1))

    def fixed(b):
        return (0, 0)

    return pl.pallas_call(
        body,
        grid=(nb + 1,),
        in_specs=[
            pl.BlockSpec((nf, _R), in_idx),
            pl.BlockSpec((de, _R), in_idx),
            pl.BlockSpec((d1, nf + de), fixed),
            pl.BlockSpec((d1, 1), fixed),
            pl.BlockSpec((d2, d1), fixed),
            pl.BlockSpec((d2, 1), fixed),
            pl.BlockSpec((do, d2), fixed),
            pl.BlockSpec((do, 1), fixed),
        ],
        out_specs=pl.BlockSpec((do, _C, _CHUNK), lambda b: (0, b, 0)),
        out_shape=jax.ShapeDtypeStruct((do, (nb + 1) * _C, _CHUNK),
                                       jnp.float32),
        scratch_shapes=[pltpu.VMEM((do, _CHUNK), jnp.float32),
                        pltpu.VMEM((do, _C), jnp.float32)],
        compiler_params=pltpu.CompilerParams(
            dimension_semantics=("arbitrary",)),
        interpret=interpret,
    )(xt, et, w1t, b1c, w2t, b2c, w3t, b3c)


_SEG_PER_W = 512
_IDX_CHUNK = 128
_PTR_CHUNKS = 5
_NPTR = _PTR_CHUNKS * _IDX_CHUNK
_DO = 16


def _sc_seg_body(nc, mtot, bseg, s_hbm, ptrb_hbm, ptra_hbm, out_hbm,
                 pvb, pva, idxb, idxa, gb, ga, out_v, sem):
    wid = lax.axis_index("s") * nc + lax.axis_index("c")
    base = wid * _SEG_PER_W
    for k in range(_PTR_CHUNKS):
        pltpu.sync_copy(ptrb_hbm.at[pl.ds(base + k * _IDX_CHUNK, _IDX_CHUNK)],
                        pvb.at[pl.ds(k * _IDX_CHUNK, _IDX_CHUNK)])
        pltpu.sync_copy(ptra_hbm.at[pl.ds(base + k * _IDX_CHUNK, _IDX_CHUNK)],
                        pva.at[pl.ds(k * _IDX_CHUNK, _IDX_CHUNK)])

    def build_f(f, carry):
        def build_t(t, c2):
            idxb[pl.ds(f * _NPTR + t * 16, 16)] = pvb[pl.ds(t * 16, 16)] \
                + f * mtot
            idxa[pl.ds(f * _NPTR + t * 16, 16)] = pva[pl.ds(t * 16, 16)] \
                + f * mtot
            return c2
        return lax.fori_loop(0, _NPTR // 16, build_t, carry)
    lax.fori_loop(0, _DO, build_f, 0)

    def gather_f(f, carry):
        cps = []
        for k in range(_PTR_CHUNKS):
            q = f * _PTR_CHUNKS + k
            cps.append(pltpu.async_copy(
                s_hbm.at[idxb.at[pl.ds(q * _IDX_CHUNK, _IDX_CHUNK)]],
                gb.at[pl.ds(q * _IDX_CHUNK, _IDX_CHUNK)], sem))
            cps.append(pltpu.async_copy(
                s_hbm.at[idxa.at[pl.ds(q * _IDX_CHUNK, _IDX_CHUNK)]],
                ga.at[pl.ds(q * _IDX_CHUNK, _IDX_CHUNK)], sem))
        for cp in cps:
            cp.wait()
        return carry
    lax.fori_loop(0, _DO, gather_f, 0)

    def diff_f(f, carry):
        def diff_v(v, c2):
            out_v[pl.ds(f * _SEG_PER_W + v * 16, 16)] = \
                ga[pl.ds(f * _NPTR + v * 16, 16)] \
                - gb[pl.ds(f * _NPTR + v * 16, 16)]
            return c2
        return lax.fori_loop(0, _SEG_PER_W // 16, diff_v, carry)
    lax.fori_loop(0, _DO, diff_f, 0)

    for f in range(_DO):
        pltpu.sync_copy(out_v.at[pl.ds(f * _SEG_PER_W, _SEG_PER_W)],
                        out_hbm.at[pl.ds(f * bseg + base, _SEG_PER_W)])


def _sc_seg_call(s_flat, ptr_pad, ptr1_pad, bseg, mtot):
    info = plsc.get_sparse_core_info()
    nc = info.num_cores
    mesh = plsc.VectorSubcoreMesh(core_axis_name="c", subcore_axis_name="s")
    fn = pl.kernel(
        functools.partial(_sc_seg_body, nc, mtot, bseg),
        mesh=mesh,
        out_type=jax.ShapeDtypeStruct((_DO * bseg,), jnp.float32),
        scratch_types=[
            pltpu.VMEM((_NPTR,), jnp.int32),
            pltpu.VMEM((_NPTR,), jnp.int32),
            pltpu.VMEM((_DO * _NPTR,), jnp.int32),
            pltpu.VMEM((_DO * _NPTR,), jnp.int32),
            pltpu.VMEM((_DO * _NPTR,), jnp.float32),
            pltpu.VMEM((_DO * _NPTR,), jnp.float32),
            pltpu.VMEM((_DO * _SEG_PER_W,), jnp.float32),
            pltpu.SemaphoreType.DMA,
        ],
    )
    return fn(s_flat, ptr_pad, ptr1_pad)


def kernel(x, node_embeddings, ptr, W1, b1, W2, b2, W3, b3):
    n = x.shape[0]
    nf = x.shape[1]
    d1 = W1.shape[1]
    d2 = W2.shape[1]
    do = W3.shape[1]
    bseg = ptr.shape[0] - 1
    s3 = _prefix_call(x.T, node_embeddings.T, W1.T, b1.reshape(d1, 1),
                      W2.T, b2.reshape(d2, 1), W3.T, b3.reshape(do, 1))
    mtot = s3.shape[1] * s3.shape[2]
    s_flat = s3.reshape(do * mtot)
    nw = bseg // _SEG_PER_W
    pad_to = (nw - 1) * _SEG_PER_W + _NPTR
    ptr_pad = jnp.concatenate(
        [ptr, jnp.full((pad_to - ptr.shape[0],), n, ptr.dtype)])
    ptr1_pad = jnp.concatenate(
        [ptr[1:], jnp.full((pad_to - ptr.shape[0] + 1,), n, ptr.dtype)])
    out = _sc_seg_call(s_flat, ptr_pad, ptr1_pad, bseg, mtot)
    return out.reshape(do, bseg).T


# speedup vs baseline: 1.2677x; 1.2677x over previous
import functools

import jax
import jax.numpy as jnp
from jax import lax
from jax.experimental import pallas as pl
from jax.experimental.pallas import tpu as pltpu
from jax.experimental.pallas import tpu_sc as plsc

_R = 25600
_CHUNK = 128
_C = _R // _CHUNK


def _prefix_body(nb, xt_ref, et_ref, w1t_ref, b1_ref, w2t_ref, b2_ref,
                 out_ref, carry_ref, cs_ref):
    b = pl.program_id(0)

    @pl.when(b == 0)
    def _():
        carry_ref[...] = jnp.zeros_like(carry_ref)

    xt = xt_ref[...]
    et = et_ref[...]
    nf = xt.shape[0]
    w1at = w1t_ref[:, 0:nf]
    w1bt = w1t_ref[:, nf:]
    h1 = jnp.maximum(
        jnp.dot(w1at, xt, preferred_element_type=jnp.float32)
        + jnp.dot(w1bt, et, preferred_element_type=jnp.float32)
        + b1_ref[...], 0.0)
    h2 = jnp.maximum(
        jnp.dot(w2t_ref[...], h1, preferred_element_type=jnp.float32)
        + b2_ref[...], 0.0)

    ri = lax.broadcasted_iota(jnp.int32, (_CHUNK, _CHUNK), 0)
    ci = lax.broadcasted_iota(jnp.int32, (_CHUNK, _CHUNK), 1)
    upper = (ri < ci).astype(jnp.float32)

    for c in range(_C):
        chunk = h2[:, c * _CHUNK:(c + 1) * _CHUNK]
        exc = jnp.dot(chunk, upper, preferred_element_type=jnp.float32)
        out_ref[:, c, :] = exc
        cs_ref[:, c:c + 1] = (exc[:, _CHUNK - 1:_CHUNK]
                              + chunk[:, _CHUNK - 1:_CHUNK])
    cs = cs_ref[...]

    rc = lax.broadcasted_iota(jnp.int32, (_C, _C), 0)
    cc = lax.broadcasted_iota(jnp.int32, (_C, _C), 1)
    upc = (rc < cc).astype(jnp.float32)
    coff = (jnp.dot(cs, upc, preferred_element_type=jnp.float32)
            + carry_ref[:, 0:1])

    for c in range(_C):
        out_ref[:, c, :] += coff[:, c:c + 1]
    carry_ref[:, 0:1] = coff[:, _C - 1:_C] + cs[:, _C - 1:_C]


def _prefix_call(xt, et, w1t, b1c, w2t, b2c, interpret=False):
    nf, n = xt.shape
    de = et.shape[0]
    d1 = w1t.shape[0]
    d2 = w2t.shape[0]
    nb = n // _R
    assert n % _R == 0
    body = functools.partial(_prefix_body, nb)

    def in_idx(b):
        return (0, jnp.minimum(b, nb - 1))

    def fixed(b):
        return (0, 0)

    return pl.pallas_call(
        body,
        grid=(nb + 1,),
        in_specs=[
            pl.BlockSpec((nf, _R), in_idx),
            pl.BlockSpec((de, _R), in_idx),
            pl.BlockSpec((d1, nf + de), fixed),
            pl.BlockSpec((d1, 1), fixed),
            pl.BlockSpec((d2, d1), fixed),
            pl.BlockSpec((d2, 1), fixed),
        ],
        out_specs=pl.BlockSpec((d2, _C, _CHUNK), lambda b: (0, b, 0)),
        out_shape=jax.ShapeDtypeStruct((d2, (nb + 1) * _C, _CHUNK),
                                       jnp.float32),
        scratch_shapes=[pltpu.VMEM((d2, _CHUNK), jnp.float32),
                        pltpu.VMEM((d2, _C), jnp.float32)],
        compiler_params=pltpu.CompilerParams(
            dimension_semantics=("arbitrary",)),
        interpret=interpret,
    )(xt, et, w1t, b1c, w2t, b2c)


_SEG_PER_W = 512
_IDX_CHUNK = 128
_PTR_CHUNKS = 5
_NPTR = _PTR_CHUNKS * _IDX_CHUNK
_DF = 8
_DOUT = 16


def _sc_seg_body(nc, mtot, bseg, s_hbm, ptrb_hbm, ptra_hbm, w3b_hbm, b3b_hbm,
                 out_hbm, pvb, pva, idxb, idxa, gb, ga, w3v, b3v, dv, cntv,
                 out_v, sem):
    wid = lax.axis_index("s") * nc + lax.axis_index("c")
    base = wid * _SEG_PER_W
    pltpu.sync_copy(w3b_hbm, w3v)
    pltpu.sync_copy(b3b_hbm, b3v)
    for k in range(_PTR_CHUNKS):
        pltpu.sync_copy(ptrb_hbm.at[pl.ds(base + k * _IDX_CHUNK, _IDX_CHUNK)],
                        pvb.at[pl.ds(k * _IDX_CHUNK, _IDX_CHUNK)])
        pltpu.sync_copy(ptra_hbm.at[pl.ds(base + k * _IDX_CHUNK, _IDX_CHUNK)],
                        pva.at[pl.ds(k * _IDX_CHUNK, _IDX_CHUNK)])

    def build_f(f, carry):
        def build_t(t, c2):
            idxb[pl.ds(f * _NPTR + t * 16, 16)] = pvb[pl.ds(t * 16, 16)] \
                + f * mtot
            idxa[pl.ds(f * _NPTR + t * 16, 16)] = pva[pl.ds(t * 16, 16)] \
                + f * mtot
            return c2
        return lax.fori_loop(0, _NPTR // 16, build_t, carry)
    lax.fori_loop(0, _DF, build_f, 0)

    def gather_f(f, carry):
        cps = []
        for k in range(_PTR_CHUNKS):
            q = f * _PTR_CHUNKS + k
            cps.append(pltpu.async_copy(
                s_hbm.at[idxb.at[pl.ds(q * _IDX_CHUNK, _IDX_CHUNK)]],
                gb.at[pl.ds(q * _IDX_CHUNK, _IDX_CHUNK)], sem))
            cps.append(pltpu.async_copy(
                s_hbm.at[idxa.at[pl.ds(q * _IDX_CHUNK, _IDX_CHUNK)]],
                ga.at[pl.ds(q * _IDX_CHUNK, _IDX_CHUNK)], sem))
        for cp in cps:
            cp.wait()
        return carry
    lax.fori_loop(0, _DF, gather_f, 0)

    def diff_f(f, carry):
        def diff_v(v, c2):
            dv[pl.ds(f * _SEG_PER_W + v * 16, 16)] = \
                ga[pl.ds(f * _NPTR + v * 16, 16)] \
                - gb[pl.ds(f * _NPTR + v * 16, 16)]
            return c2
        return lax.fori_loop(0, _SEG_PER_W // 16, diff_v, carry)
    lax.fori_loop(0, _DF, diff_f, 0)

    def cnt_v(v, carry):
        cntv[pl.ds(v * 16, 16)] = (
            pva[pl.ds(v * 16, 16)] - pvb[pl.ds(v * 16, 16)]
        ).astype(jnp.float32)
        return carry
    lax.fori_loop(0, _SEG_PER_W // 16, cnt_v, 0)

    def out_f(fo, carry):
        def out_vv(v, c2):
            acc = b3v[pl.ds(fo * 16, 16)] * cntv[pl.ds(v * 16, 16)]
            for k in range(_DF):
                acc = acc + w3v[pl.ds(k * _DOUT * 16 + fo * 16, 16)] \
                    * dv[pl.ds(k * _SEG_PER_W + v * 16, 16)]
            out_v[pl.ds(fo * _SEG_PER_W + v * 16, 16)] = acc
            return c2
        return lax.fori_loop(0, _SEG_PER_W // 16, out_vv, carry)
    lax.fori_loop(0, _DOUT, out_f, 0)

    for f in range(_DOUT):
        pltpu.sync_copy(out_v.at[pl.ds(f * _SEG_PER_W, _SEG_PER_W)],
                        out_hbm.at[pl.ds(f * bseg + base, _SEG_PER_W)])


def _sc_seg_call(s_flat, ptr_pad, ptr1_pad, w3b, b3b, bseg, mtot):
    info = plsc.get_sparse_core_info()
    nc = info.num_cores
    mesh = plsc.VectorSubcoreMesh(core_axis_name="c", subcore_axis_name="s")
    fn = pl.kernel(
        functools.partial(_sc_seg_body, nc, mtot, bseg),
        mesh=mesh,
        out_type=jax.ShapeDtypeStruct((_DOUT * bseg,), jnp.float32),
        scratch_types=[
            pltpu.VMEM((_NPTR,), jnp.int32),
            pltpu.VMEM((_NPTR,), jnp.int32),
            pltpu.VMEM((_DF * _NPTR,), jnp.int32),
            pltpu.VMEM((_DF * _NPTR,), jnp.int32),
            pltpu.VMEM((_DF * _NPTR,), jnp.float32),
            pltpu.VMEM((_DF * _NPTR,), jnp.float32),
            pltpu.VMEM((_DF * _DOUT * 16,), jnp.float32),
            pltpu.VMEM((_DOUT * 16,), jnp.float32),
            pltpu.VMEM((_DF * _SEG_PER_W,), jnp.float32),
            pltpu.VMEM((_SEG_PER_W,), jnp.float32),
            pltpu.VMEM((_DOUT * _SEG_PER_W,), jnp.float32),
            pltpu.SemaphoreType.DMA,
        ],
    )
    return fn(s_flat, ptr_pad, ptr1_pad, w3b, b3b)


def kernel(x, node_embeddings, ptr, W1, b1, W2, b2, W3, b3):
    n = x.shape[0]
    nf = x.shape[1]
    d1 = W1.shape[1]
    d2 = W2.shape[1]
    do = W3.shape[1]
    bseg = ptr.shape[0] - 1
    s3 = _prefix_call(x.T, node_embeddings.T, W1.T, b1.reshape(d1, 1),
                      W2.T, b2.reshape(d2, 1))
    mtot = s3.shape[1] * s3.shape[2]
    s_flat = s3.reshape(d2 * mtot)
    w3b = jnp.broadcast_to(W3[:, :, None], (d2, do, 16)).reshape(-1)
    b3b = jnp.broadcast_to(b3[:, None], (do, 16)).reshape(-1)
    nw = bseg // _SEG_PER_W
    pad_to = (nw - 1) * _SEG_PER_W + _NPTR
    ptr_pad = jnp.concatenate(
        [ptr, jnp.full((pad_to - ptr.shape[0],), n, ptr.dtype)])
    ptr1_pad = jnp.concatenate(
        [ptr[1:], jnp.full((pad_to - ptr.shape[0] + 1,), n, ptr.dtype)])
    out = _sc_seg_call(s_flat, ptr_pad, ptr1_pad, w3b, b3b, bseg, mtot)
    return out.reshape(do, bseg).T
